# baseline (speedup 1.0000x reference)
import dataclasses
import functools

import jax
import jax.numpy as jnp
from jax import lax
from jax.experimental import pallas as pl
from jax.experimental.pallas import tpu as pltpu
from jax.experimental.pallas import tpu_sc as plsc

NC, NS = 2, 16
NW = NC * NS
CHUNK = 1280
NSTREAM = 4
SUB = CHUNK // NSTREAM


def kernel(x, weight):
    batch, seq = x.shape
    num_idx = batch * seq
    dim = weight.shape[1]
    b_per_w = num_idx // NW
    n_ch = b_per_w // CHUNK
    idx_flat = x.reshape(num_idx)

    mesh = plsc.VectorSubcoreMesh(core_axis_name="c", subcore_axis_name="s")

    cp = pltpu.CompilerParams(use_tc_tiling_on_sc=False)
    if "needs_layout_passes" in pltpu.CompilerParams.__dataclass_fields__:
        cp = dataclasses.replace(cp, needs_layout_passes=False)

    @functools.partial(
        pl.kernel,
        mesh=mesh,
        compiler_params=cp,
        out_type=jax.ShapeDtypeStruct((num_idx, dim), jnp.float32),
        scratch_types=[
            pltpu.VMEM((CHUNK,), jnp.int32),
            pltpu.VMEM((CHUNK,), jnp.int32),
            pltpu.VMEM((CHUNK, dim), jnp.float32),
            pltpu.VMEM((CHUNK, dim), jnp.float32),
            pltpu.SemaphoreType.DMA,
            pltpu.SemaphoreType.DMA,
        ],
    )
    def gather_kernel(table_hbm, idx_hbm, out_hbm, i0, i1, r0, r1, sg0, sg1):
        wid = lax.axis_index("s") * NC + lax.axis_index("c")
        base = wid * b_per_w

        def fire(iv, rv, sem):
            for s in range(NSTREAM):
                pltpu.async_copy(table_hbm.at[iv.at[pl.ds(s * SUB, SUB)]],
                                 rv.at[pl.ds(s * SUB, SUB)], sem)

        def drain(iv, rv, sem):
            for s in range(NSTREAM):
                pltpu.make_async_copy(table_hbm.at[iv.at[pl.ds(s * SUB, SUB)]],
                                      rv.at[pl.ds(s * SUB, SUB)], sem).wait()

        pltpu.sync_copy(idx_hbm.at[pl.ds(base, CHUNK)], i0)
        fire(i0, r0, sg0)

        @pl.loop(0, n_ch, step=2)
        def _(t):
            off1 = base + (t + 1) * CHUNK
            pltpu.sync_copy(idx_hbm.at[pl.ds(off1, CHUNK)], i1)
            fire(i1, r1, sg1)

            drain(i0, r0, sg0)
            pltpu.sync_copy(r0, out_hbm.at[pl.ds(base + t * CHUNK, CHUNK)])

            @pl.when(t + 2 < n_ch)
            def _():
                off2 = base + (t + 2) * CHUNK
                pltpu.sync_copy(idx_hbm.at[pl.ds(off2, CHUNK)], i0)
                fire(i0, r0, sg0)

            drain(i1, r1, sg1)
            pltpu.sync_copy(r1, out_hbm.at[pl.ds(off1, CHUNK)])

    out = gather_kernel(weight, idx_flat)
    return out.reshape(batch, seq, dim)


# speedup vs baseline: 1.0015x; 1.0015x over previous
import dataclasses
import functools

import jax
import jax.numpy as jnp
from jax import lax
from jax.experimental import pallas as pl
from jax.experimental.pallas import tpu as pltpu
from jax.experimental.pallas import tpu_sc as plsc

NC, NS = 2, 16
NW = NC * NS
CHUNK = 1600


def kernel(x, weight):
    batch, seq = x.shape
    num_idx = batch * seq
    dim = weight.shape[1]
    b_per_w = num_idx // NW
    n_ch = b_per_w // CHUNK
    bat_per_ch = CHUNK // seq


    mesh = plsc.VectorSubcoreMesh(core_axis_name="c", subcore_axis_name="s")

    cp = pltpu.CompilerParams(use_tc_tiling_on_sc=False)
    if "needs_layout_passes" in pltpu.CompilerParams.__dataclass_fields__:
        cp = dataclasses.replace(cp, needs_layout_passes=False)

    @functools.partial(
        pl.kernel,
        mesh=mesh,
        compiler_params=cp,
        out_type=jax.ShapeDtypeStruct((num_idx, dim), jnp.float32),
        scratch_types=[
            pltpu.VMEM((CHUNK // 50, 50), jnp.int32),
            pltpu.VMEM((CHUNK // 50, 50), jnp.int32),
            pltpu.VMEM((CHUNK, dim), jnp.float32),
            pltpu.VMEM((CHUNK, dim), jnp.float32),
            pltpu.SemaphoreType.DMA,
            pltpu.SemaphoreType.DMA,
        ],
    )
    def gather_kernel(table_hbm, idx_hbm, out_hbm, i0, i1, r0, r1, sg0, sg1):
        wid = lax.axis_index("s") * NC + lax.axis_index("c")
        base = wid * b_per_w

        bbase = wid * (b_per_w // seq)

        def load_idx(t, iv):
            pltpu.sync_copy(
                idx_hbm.at[pl.ds(bbase + t * bat_per_ch, bat_per_ch)], iv)

        def fire(iv, rv, sem):
            for b in range(bat_per_ch):
                pltpu.async_copy(table_hbm.at[iv.at[b]],
                                 rv.at[pl.ds(b * seq, seq)], sem)

        def drain(iv, rv, sem):
            for b in range(bat_per_ch):
                pltpu.make_async_copy(table_hbm.at[iv.at[b]],
                                      rv.at[pl.ds(b * seq, seq)], sem).wait()

        load_idx(0, i0)
        fire(i0, r0, sg0)

        @pl.loop(0, n_ch, step=2)
        def _(t):
            off1 = base + (t + 1) * CHUNK
            load_idx(t + 1, i1)
            fire(i1, r1, sg1)

            drain(i0, r0, sg0)
            pltpu.sync_copy(r0, out_hbm.at[pl.ds(base + t * CHUNK, CHUNK)])

            @pl.when(t + 2 < n_ch)
            def _():
                load_idx(t + 2, i0)
                fire(i0, r0, sg0)

            drain(i1, r1, sg1)
            pltpu.sync_copy(r1, out_hbm.at[pl.ds(off1, CHUNK)])

    out = gather_kernel(weight, x)
    return out.reshape(batch, seq, dim)
